# baseline (speedup 1.0000x reference)
import functools

import jax
import jax.numpy as jnp
from jax import lax
from jax.experimental import pallas as pl
from jax.experimental.pallas import tpu as pltpu
from jax.experimental.pallas import tpu_sc as plsc

EMB = 64
SEQ = 200
CSEQ = 2
SPLITS = ((0, 128), (128, 72))
NBUF = 2


@jax.jit
def _gather_rows(positions, table):
    info = plsc.get_sparse_core_info()
    nw = info.num_cores * info.num_subcores
    n_seq, seq = positions.shape
    per_w = n_seq // nw
    n_chunks = per_w // CSEQ
    n_outer = n_chunks // NBUF
    n_tab = table.shape[0]
    mesh = plsc.VectorSubcoreMesh(core_axis_name="c", subcore_axis_name="s")

    @functools.partial(
        pl.kernel,
        mesh=mesh,
        out_type=jax.ShapeDtypeStruct((n_seq, seq, EMB), jnp.float32),
        compiler_params=pltpu.CompilerParams(use_tc_tiling_on_sc=False),
        scratch_types=[
            pltpu.VMEM((NBUF, CSEQ, SEQ), jnp.int32),
            pltpu.VMEM((NBUF, CSEQ, SEQ, EMB), jnp.float32),
            pltpu.VMEM_SHARED((n_tab, EMB), jnp.float32),
            pltpu.SemaphoreType.DMA,
            pltpu.SemaphoreType.DMA,
            pltpu.SemaphoreType.DMA,
            pltpu.SemaphoreType.DMA,
            pltpu.SemaphoreType.DMA,
            pltpu.SemaphoreType.DMA,
        ],
    )
    def k(idx_hbm, table_hbm, out_hbm, idx_v, rows_v, table_sp,
          si0, si1, sg0, sg1, ss0, ss1):
        wid = lax.axis_index("s") * info.num_cores + lax.axis_index("c")
        seq0 = wid * per_w
        sem_i = (si0, si1)
        sem_g = (sg0, sg1)
        sem_s = (ss0, ss1)

        @pl.when(lax.axis_index("s") == 0)
        def _stage():
            pltpu.sync_copy(table_hbm, table_sp)

        plsc.subcore_barrier()

        def fire_idx(ci, b):
            cj = jnp.minimum(ci, n_chunks - 1)
            pltpu.async_copy(
                idx_hbm.at[pl.ds(seq0 + cj * CSEQ, CSEQ), :], idx_v.at[b], sem_i[b]
            )

        def wait_idx(b):
            pltpu.make_async_copy(
                idx_hbm.at[pl.ds(0, CSEQ), :], idx_v.at[b], sem_i[b]
            ).wait()

        def gather_and_store(ci, b):
            wait_idx(b)
            handles = []
            for s in range(CSEQ):
                for off, ln in SPLITS:
                    handles.append(pltpu.async_copy(
                        table_sp.at[idx_v.at[b].at[s, pl.ds(off, ln)]],
                        rows_v.at[b].at[s, pl.ds(off, ln), :],
                        sem_g[b],
                    ))
            for h in handles:
                h.wait()
            fire_idx(ci + NBUF, b)
            pltpu.async_copy(
                rows_v.at[b], out_hbm.at[pl.ds(seq0 + ci * CSEQ, CSEQ), :, :], sem_s[b]
            )

        def wait_store(b):
            pltpu.make_async_copy(
                rows_v.at[b], out_hbm.at[pl.ds(0, CSEQ), :, :], sem_s[b]
            ).wait()

        for b in range(NBUF):
            fire_idx(b, b)
        for b in range(NBUF):
            gather_and_store(b, b)

        def body(g, carry):
            for b in range(NBUF):
                ci = g * NBUF + b
                wait_store(b)
                gather_and_store(ci, b)
            return carry

        lax.fori_loop(1, n_outer, body, 0)

        for b in range(NBUF):
            wait_store(b)
            wait_idx(b)

    return k(positions, table)


def kernel(positions, weights):
    out = _gather_rows(positions.astype(jnp.int32), weights)
    return lax.stop_gradient(out)
